# Initial kernel scaffold; baseline (speedup 1.0000x reference)
import jax
import jax.numpy as jnp
from jax.experimental import pallas as pl


def kernel(idx, targets, table):
    raise NotImplementedError("write your pallas kernel here")


# speedup vs baseline: 1.4246x; 1.4246x over previous
import functools

import jax
import jax.numpy as jnp
from jax import lax
from jax.experimental import pallas as pl
from jax.experimental.pallas import tpu as pltpu
from jax.experimental.pallas import tpu_sc as plsc

VOCAB = 1000
NC, NS, LANES = 2, 16, 16
NW = NC * NS
BT = 1024 * 50
PER_W = BT // NW
CHUNK = 32
NCHUNK = PER_W // CHUNK


def _lse_body(tab_ref, out_ref):
    x = tab_ref[...]
    m = jnp.max(x, axis=1)
    s = jnp.sum(jnp.exp(x - m[:, None]), axis=1)
    out_ref[...] = m + jnp.log(s)


_sc_mesh = plsc.VectorSubcoreMesh(core_axis_name="c", subcore_axis_name="s",
                                  num_cores=NC, num_subcores=NS)


@functools.partial(
    pl.kernel,
    out_type=(
        jax.ShapeDtypeStruct((BT, VOCAB), jnp.float32),
        jax.ShapeDtypeStruct((NW, LANES), jnp.float32),
    ),
    mesh=_sc_mesh,
    compiler_params=pltpu.CompilerParams(use_tc_tiling_on_sc=False,
                                         needs_layout_passes=False),
    scratch_types=[
        pltpu.VMEM((PER_W,), jnp.int32),
        pltpu.VMEM((PER_W,), jnp.int32),
        pltpu.VMEM((VOCAB,), jnp.float32),
        pltpu.VMEM((CHUNK, VOCAB), jnp.float32),
        pltpu.VMEM((CHUNK, VOCAB), jnp.float32),
        pltpu.VMEM((LANES,), jnp.float32),
        pltpu.SemaphoreType.DMA,
        pltpu.SemaphoreType.DMA,
    ],
)
def _sc_gather_loss(table_hbm, idx_hbm, tgt_hbm, lse_hbm,
                    out_hbm, part_hbm,
                    idx_v, tgt_v, lse_v, buf0, buf1, acc_v, sem0, sem1):
    wid = lax.axis_index("s") * NC + lax.axis_index("c")
    base = wid * PER_W
    pltpu.sync_copy(idx_hbm.at[pl.ds(base, PER_W)], idx_v)
    pltpu.sync_copy(tgt_hbm.at[pl.ds(base, PER_W)], tgt_v)
    pltpu.sync_copy(lse_hbm, lse_v)
    acc_v[...] = jnp.zeros((LANES,), jnp.float32)

    bufs = (buf0, buf1)
    sems = (sem0, sem1)

    def gather_desc(c, b):
        return pltpu.make_async_copy(
            table_hbm.at[idx_v.at[pl.ds(c * CHUNK, CHUNK)]], bufs[b], sems[b])

    pltpu.async_copy(table_hbm.at[idx_v.at[pl.ds(0, CHUNK)]], buf0, sem0)

    @pl.loop(0, NCHUNK, step=2)
    def _chunks(j):
        for b in range(2):
            c = j + b
            gather_desc(c, b).wait()
            nxt = c + 1

            @pl.when(nxt < NCHUNK)
            def _():
                pltpu.async_copy(
                    table_hbm.at[idx_v.at[pl.ds(nxt * CHUNK, CHUNK)]],
                    bufs[1 - b], sems[1 - b])

            for h in range(CHUNK // LANES):
                off = c * CHUNK + h * LANES
                tg = tgt_v[pl.ds(off, LANES)]
                ix = idx_v[pl.ds(off, LANES)]
                rowid = lax.iota(jnp.int32, LANES) + (h * LANES)
                tgt_logit = plsc.load_gather(bufs[b], [rowid, tg])
                lse_val = plsc.load_gather(lse_v, [ix])
                acc_v[...] = acc_v[...] + (lse_val - tgt_logit)

            pltpu.sync_copy(bufs[b], out_hbm.at[pl.ds(base + c * CHUNK, CHUNK)])

    pltpu.sync_copy(acc_v, part_hbm.at[wid])


@jax.jit
def kernel(idx, targets, table):
    Bb, Tt = idx.shape
    flat_idx = idx.reshape(-1).astype(jnp.int32)
    flat_tgt = targets.reshape(-1).astype(jnp.int32)
    table = table.astype(jnp.float32)
    lse = pl.pallas_call(
        _lse_body,
        out_shape=jax.ShapeDtypeStruct((VOCAB,), jnp.float32),
    )(table)
    logits_flat, parts = _sc_gather_loss(table, flat_idx, flat_tgt, lse)
    loss = jnp.sum(parts) / jnp.float32(BT)
    return logits_flat.reshape(Bb, Tt, VOCAB), loss
